# baseline (speedup 1.0000x reference)
import functools
import math

import jax
import jax.numpy as jnp
from jax.experimental import pallas as pl
from jax.experimental.pallas import tpu as pltpu

NUM_EXPERTS = 64
D_MODEL = 1024
D_FF = 1024
NUM_TOKENS = 2048
CAP = 32


def _router_body(x_ref, wr_ref, te_ref, wt_ref):
    lg = jnp.dot(x_ref[...], wr_ref[...], preferred_element_type=jnp.float32)
    mx = jnp.max(lg, axis=1, keepdims=True)
    s = jnp.sum(jnp.exp(lg - mx), axis=1, keepdims=True)
    lane = jax.lax.broadcasted_iota(jnp.int32, lg.shape, 1)
    te_ref[...] = jnp.min(jnp.where(lg == mx, lane, NUM_EXPERTS), axis=1,
                          keepdims=True)
    wt_ref[...] = 1.0 / s


def _moe_body(idx_ref, wts_ref, x_ref, w1_ref, b1_ref, w2_ref, b2_ref,
              out_ref, xe_ref):
    e = pl.program_id(0)

    @pl.when(e == 0)
    def _init():
        out_ref[...] = jnp.zeros_like(out_ref)

    for c in range(CAP):
        t = idx_ref[e, c]
        ts = jnp.where(t >= NUM_TOKENS, 0, t)
        xe_ref[pl.ds(c, 1), :] = x_ref[pl.ds(ts, 1), :]

    h = jnp.maximum(
        jnp.dot(xe_ref[...], w1_ref[0], preferred_element_type=jnp.float32)
        + b1_ref[0], 0.0)
    y = (jnp.dot(h, w2_ref[0], preferred_element_type=jnp.float32)
         + b2_ref[0])

    for c in range(CAP):
        t = idx_ref[e, c]

        @pl.when(t < NUM_TOKENS)
        def _store():
            out_ref[pl.ds(t, 1), :] = y[c:c + 1, :] * wts_ref[e, c]


@jax.jit
def kernel(x, w_router, w1, b1, w2, b2, rng):
    T, D = x.shape
    E = NUM_EXPERTS

    te_col, wt_col = pl.pallas_call(
        _router_body,
        out_shape=[
            jax.ShapeDtypeStruct((T, 1), jnp.int32),
            jax.ShapeDtypeStruct((T, 1), jnp.float32),
        ],
    )(x, w_router)
    top_e = te_col[:, 0]
    wt = wt_col[:, 0]

    onehot = (jnp.arange(E, dtype=jnp.int32)[None, :] == top_e[:, None])
    pos = jnp.cumsum(onehot.astype(jnp.int32), axis=0) - 1
    pos_t = jnp.take_along_axis(pos, top_e[:, None], axis=1)[:, 0]
    slot = jnp.where(pos_t < CAP, top_e * CAP + pos_t, E * CAP)
    tok_ids = jnp.arange(T, dtype=jnp.int32)
    slot_to_token = jnp.full((E * CAP,), T, jnp.int32).at[slot].set(
        tok_ids, mode="drop", unique_indices=True).reshape(E, CAP)
    slot_wt = jnp.zeros((E * CAP,), jnp.float32).at[slot].set(
        wt, mode="drop", unique_indices=True).reshape(E, CAP)

    grid_spec = pltpu.PrefetchScalarGridSpec(
        num_scalar_prefetch=2,
        grid=(E,),
        in_specs=[
            pl.BlockSpec((T, D), lambda e, *_: (0, 0)),
            pl.BlockSpec((1, D, D_FF), lambda e, *_: (e, 0, 0)),
            pl.BlockSpec((1, 1, D_FF), lambda e, *_: (e, 0, 0)),
            pl.BlockSpec((1, D_FF, D), lambda e, *_: (e, 0, 0)),
            pl.BlockSpec((1, 1, D), lambda e, *_: (e, 0, 0)),
        ],
        out_specs=pl.BlockSpec((T, D), lambda e, *_: (0, 0)),
        scratch_shapes=[pltpu.VMEM((CAP, D), jnp.float32)],
    )
    out = pl.pallas_call(
        _moe_body,
        grid_spec=grid_spec,
        out_shape=jax.ShapeDtypeStruct((T, D), x.dtype),
        compiler_params=pltpu.CompilerParams(
            dimension_semantics=("arbitrary",),
        ),
    )(slot_to_token, slot_wt, x, w1, b1.reshape(E, 1, D_FF), w2,
      b2.reshape(E, 1, D))
    return out


# speedup vs baseline: 1.1822x; 1.1822x over previous
import functools
import math

import jax
import jax.numpy as jnp
from jax import lax
from jax.experimental import pallas as pl
from jax.experimental.pallas import tpu as pltpu
from jax.experimental.pallas import tpu_sc as plsc

NUM_EXPERTS = 64
D_MODEL = 1024
D_FF = 1024
NUM_TOKENS = 2048
CAP = 32
PAD_LEN = 32


def _router_body(x_ref, wr_ref, te_ref, wt_ref):
    lg = jnp.dot(x_ref[...], wr_ref[...], preferred_element_type=jnp.float32)
    mx = jnp.max(lg, axis=1, keepdims=True)
    s = jnp.sum(jnp.exp(lg - mx), axis=1, keepdims=True)
    lane = jax.lax.broadcasted_iota(jnp.int32, lg.shape, 1)
    te_ref[...] = jnp.min(jnp.where(lg == mx, lane, NUM_EXPERTS), axis=1,
                          keepdims=True)
    wt_ref[...] = 1.0 / s


def _compact_body(te_hbm, stt_hbm, te_v, stt_v, pad_v):
    L = 16
    PAD = 8
    wid = lax.axis_index("s") * 2 + lax.axis_index("c")
    pltpu.sync_copy(te_hbm, te_v)

    zeros = jnp.zeros((L,), jnp.int32)
    ones = jnp.ones((L,), jnp.int32)
    pad_v[pl.ds(0, L)] = zeros
    fill = jnp.full((L,), NUM_TOKENS, jnp.int32)
    for i in range(2 * CAP // L):
        stt_v[pl.ds(i * L, L)] = fill
    lane = lax.iota(jnp.int32, L)
    last = jnp.full((L,), PAD + L - 1, jnp.int32)

    def body(g, bases):
        tev = te_v[pl.ds(g * L, L)]
        tok = g * L + lane
        outs = []
        for sub in range(2):
            base = bases[sub]
            m = tev == (wid * 2 + sub)
            acc = jnp.where(m, ones, zeros)
            for k in (1, 2, 4, 8):
                pad_v[pl.ds(PAD, L)] = acc
                acc = acc + plsc.load_gather(pad_v, [lane + (PAD - k)])
            pos0 = acc - 1 + base
            ok = m & (pos0 < CAP)
            idx = jnp.where(ok, sub * CAP + pos0, zeros)
            plsc.store_scatter(stt_v, [idx], tok, mask=ok)
            pad_v[pl.ds(PAD, L)] = acc
            outs.append(base + plsc.load_gather(pad_v, [last]))
        return tuple(outs)

    lax.fori_loop(0, NUM_TOKENS // L, body, (zeros, zeros))
    pltpu.sync_copy(stt_v, stt_hbm.at[pl.ds(wid * 2 * CAP, 2 * CAP)])


def _sc_compact(top_e):
    mesh = plsc.VectorSubcoreMesh(core_axis_name="c", subcore_axis_name="s")
    f = pl.kernel(
        _compact_body,
        mesh=mesh,
        compiler_params=pltpu.CompilerParams(needs_layout_passes=False),
        out_type=jax.ShapeDtypeStruct((NUM_EXPERTS * CAP,), jnp.int32),
        scratch_types=[
            pltpu.VMEM((NUM_TOKENS,), jnp.int32),
            pltpu.VMEM((2 * CAP,), jnp.int32),
            pltpu.VMEM((PAD_LEN,), jnp.int32),
        ],
    )
    return f(top_e)


def _moe_body(idx_ref, wts_ref, x_ref, w1_ref, b1_ref, w2_ref, b2_ref,
              out_ref, xe_ref):
    e = pl.program_id(0)

    @pl.when(e == 0)
    def _init():
        out_ref[...] = jnp.zeros_like(out_ref)

    for c in range(CAP):
        t = idx_ref[e, c]
        ts = jnp.where(t >= NUM_TOKENS, 0, t)
        xe_ref[pl.ds(c, 1), :] = x_ref[pl.ds(ts, 1), :]

    h = jnp.maximum(
        jnp.dot(xe_ref[...], w1_ref[0], preferred_element_type=jnp.float32)
        + b1_ref[0], 0.0)
    y = (jnp.dot(h, w2_ref[0], preferred_element_type=jnp.float32)
         + b2_ref[0])

    for c in range(CAP):
        t = idx_ref[e, c]

        @pl.when(t < NUM_TOKENS)
        def _store():
            out_ref[pl.ds(t, 1), :] = y[c:c + 1, :] * wts_ref[t]


@jax.jit
def kernel(x, w_router, w1, b1, w2, b2, rng):
    T, D = x.shape
    E = NUM_EXPERTS

    te_col, wt_col = pl.pallas_call(
        _router_body,
        out_shape=[
            jax.ShapeDtypeStruct((T, 1), jnp.int32),
            jax.ShapeDtypeStruct((T, 1), jnp.float32),
        ],
    )(x, w_router)
    top_e = te_col[:, 0]
    wt = wt_col[:, 0]

    stt = _sc_compact(top_e)
    slot_to_token = stt.reshape(E, CAP)

    grid_spec = pltpu.PrefetchScalarGridSpec(
        num_scalar_prefetch=2,
        grid=(E,),
        in_specs=[
            pl.BlockSpec((T, D), lambda e, *_: (0, 0)),
            pl.BlockSpec((1, D, D_FF), lambda e, *_: (e, 0, 0)),
            pl.BlockSpec((1, 1, D_FF), lambda e, *_: (e, 0, 0)),
            pl.BlockSpec((1, D_FF, D), lambda e, *_: (e, 0, 0)),
            pl.BlockSpec((1, 1, D), lambda e, *_: (e, 0, 0)),
        ],
        out_specs=pl.BlockSpec((T, D), lambda e, *_: (0, 0)),
        scratch_shapes=[pltpu.VMEM((CAP, D), jnp.float32)],
    )
    out = pl.pallas_call(
        _moe_body,
        grid_spec=grid_spec,
        out_shape=jax.ShapeDtypeStruct((T, D), x.dtype),
        compiler_params=pltpu.CompilerParams(
            dimension_semantics=("arbitrary",),
        ),
    )(slot_to_token, wt, x, w1, b1.reshape(E, 1, D_FF), w2,
      b2.reshape(E, 1, D))
    return out


# speedup vs baseline: 1.1896x; 1.0063x over previous
import functools
import math

import jax
import jax.numpy as jnp
from jax import lax
from jax.experimental import pallas as pl
from jax.experimental.pallas import tpu as pltpu
from jax.experimental.pallas import tpu_sc as plsc

NUM_EXPERTS = 64
D_MODEL = 1024
D_FF = 1024
NUM_TOKENS = 2048
CAP = 32
PAD_LEN = 32


def _router_body(x_ref, wr_ref, te_ref, wt_ref):
    lg = jnp.dot(x_ref[...], wr_ref[...], preferred_element_type=jnp.float32)
    mx = jnp.max(lg, axis=1, keepdims=True)
    s = jnp.sum(jnp.exp(lg - mx), axis=1, keepdims=True)
    lane = jax.lax.broadcasted_iota(jnp.int32, lg.shape, 1)
    te_ref[...] = jnp.min(jnp.where(lg == mx, lane, NUM_EXPERTS), axis=1,
                          keepdims=True)
    wt_ref[...] = 1.0 / s


def _compact_body(te_hbm, stt_hbm, te_v, stt_v, pad_v):
    L = 16
    PAD = 8
    wid = lax.axis_index("s") * 2 + lax.axis_index("c")
    pltpu.sync_copy(te_hbm, te_v)

    zeros = jnp.zeros((L,), jnp.int32)
    ones = jnp.ones((L,), jnp.int32)
    pad_v[pl.ds(0, L)] = zeros
    fill = jnp.full((L,), NUM_TOKENS, jnp.int32)
    for i in range(2 * CAP // L):
        stt_v[pl.ds(i * L, L)] = fill
    lane = lax.iota(jnp.int32, L)
    last = jnp.full((L,), PAD + L - 1, jnp.int32)

    def body(g, bases):
        tev = te_v[pl.ds(g * L, L)]
        m0 = tev == wid * 2
        m1 = tev == wid * 2 + 1

        def compact(bases):
            b0, b1 = bases
            tok = g * L + lane
            acc = (jnp.where(m0, ones, zeros)
                   + jnp.where(m1, jnp.full((L,), 256, jnp.int32), zeros))
            for k in (1, 2, 4, 8):
                pad_v[pl.ds(PAD, L)] = acc
                acc = acc + plsc.load_gather(pad_v, [lane + (PAD - k)])
            pos0 = (acc & 255) - 1 + b0
            pos1 = (acc >> 8) - 1 + b1
            ok0 = m0 & (pos0 < CAP)
            ok1 = m1 & (pos1 < CAP)
            idx = jnp.where(ok0, pos0, jnp.where(ok1, CAP + pos1, zeros))
            plsc.store_scatter(stt_v, [idx], tok, mask=ok0 | ok1)
            pad_v[pl.ds(PAD, L)] = acc
            tot = plsc.load_gather(pad_v, [last])
            return b0 + (tot & 255), b1 + (tot >> 8)

        return lax.cond(jnp.any(m0 | m1), compact, lambda b: b, bases)

    lax.fori_loop(0, NUM_TOKENS // L, body, (zeros, zeros))
    pltpu.sync_copy(stt_v, stt_hbm.at[pl.ds(wid * 2 * CAP, 2 * CAP)])


def _sc_compact(top_e):
    mesh = plsc.VectorSubcoreMesh(core_axis_name="c", subcore_axis_name="s")
    f = pl.kernel(
        _compact_body,
        mesh=mesh,
        compiler_params=pltpu.CompilerParams(needs_layout_passes=False),
        out_type=jax.ShapeDtypeStruct((NUM_EXPERTS * CAP,), jnp.int32),
        scratch_types=[
            pltpu.VMEM((NUM_TOKENS,), jnp.int32),
            pltpu.VMEM((2 * CAP,), jnp.int32),
            pltpu.VMEM((PAD_LEN,), jnp.int32),
        ],
    )
    return f(top_e)


def _moe_body(idx_ref, wts_ref, x_ref, w1_ref, b1_ref, w2_ref, b2_ref,
              out_ref, xe_ref):
    e = pl.program_id(0)

    @pl.when(e == 0)
    def _init():
        out_ref[...] = jnp.zeros_like(out_ref)

    for c in range(CAP):
        t = idx_ref[e, c]
        ts = jnp.where(t >= NUM_TOKENS, 0, t)
        xe_ref[pl.ds(c, 1), :] = x_ref[pl.ds(ts, 1), :]

    h = jnp.maximum(
        jnp.dot(xe_ref[...], w1_ref[0], preferred_element_type=jnp.float32)
        + b1_ref[0], 0.0)
    y = (jnp.dot(h, w2_ref[0], preferred_element_type=jnp.float32)
         + b2_ref[0])

    for c in range(CAP):
        t = idx_ref[e, c]

        @pl.when(t < NUM_TOKENS)
        def _store():
            out_ref[pl.ds(t, 1), :] = y[c:c + 1, :] * wts_ref[t]


@jax.jit
def kernel(x, w_router, w1, b1, w2, b2, rng):
    T, D = x.shape
    E = NUM_EXPERTS

    te_col, wt_col = pl.pallas_call(
        _router_body,
        out_shape=[
            jax.ShapeDtypeStruct((T, 1), jnp.int32),
            jax.ShapeDtypeStruct((T, 1), jnp.float32),
        ],
    )(x, w_router)
    top_e = te_col[:, 0]
    wt = wt_col[:, 0]

    stt = _sc_compact(top_e)
    slot_to_token = stt.reshape(E, CAP)

    grid_spec = pltpu.PrefetchScalarGridSpec(
        num_scalar_prefetch=2,
        grid=(E,),
        in_specs=[
            pl.BlockSpec((T, D), lambda e, *_: (0, 0)),
            pl.BlockSpec((1, D, D_FF), lambda e, *_: (e, 0, 0)),
            pl.BlockSpec((1, 1, D_FF), lambda e, *_: (e, 0, 0)),
            pl.BlockSpec((1, D_FF, D), lambda e, *_: (e, 0, 0)),
            pl.BlockSpec((1, 1, D), lambda e, *_: (e, 0, 0)),
        ],
        out_specs=pl.BlockSpec((T, D), lambda e, *_: (0, 0)),
        scratch_shapes=[pltpu.VMEM((CAP, D), jnp.float32)],
    )
    out = pl.pallas_call(
        _moe_body,
        grid_spec=grid_spec,
        out_shape=jax.ShapeDtypeStruct((T, D), x.dtype),
        compiler_params=pltpu.CompilerParams(
            dimension_semantics=("arbitrary",),
        ),
    )(slot_to_token, wt, x, w1, b1.reshape(E, 1, D_FF), w2,
      b2.reshape(E, 1, D))
    return out


# speedup vs baseline: 1.2156x; 1.0219x over previous
import functools
import math

import jax
import jax.numpy as jnp
from jax import lax
from jax.experimental import pallas as pl
from jax.experimental.pallas import tpu as pltpu
from jax.experimental.pallas import tpu_sc as plsc

NUM_EXPERTS = 64
D_MODEL = 1024
D_FF = 1024
NUM_TOKENS = 2048
CAP = 32
PAD_LEN = 32


def _router_body(x_ref, wr_ref, te_ref, wt_ref):
    lg = jnp.dot(x_ref[...], wr_ref[...], preferred_element_type=jnp.float32)
    mx = jnp.max(lg, axis=1, keepdims=True)
    s = jnp.sum(jnp.exp(lg - mx), axis=1, keepdims=True)
    lane = jax.lax.broadcasted_iota(jnp.int32, lg.shape, 1)
    te_ref[...] = jnp.min(jnp.where(lg == mx, lane, NUM_EXPERTS), axis=1,
                          keepdims=True)
    wt_ref[...] = 1.0 / s


def _compact_body(te_hbm, stt_hbm, te_v, stt_v, pad_v):
    L = 16
    PAD = 8
    wid = lax.axis_index("s") * 2 + lax.axis_index("c")
    pltpu.sync_copy(te_hbm, te_v)

    zeros = jnp.zeros((L,), jnp.int32)
    ones = jnp.ones((L,), jnp.int32)
    pad_v[pl.ds(0, L)] = zeros
    fill = jnp.full((L,), NUM_TOKENS, jnp.int32)
    for i in range(2 * CAP // L):
        stt_v[pl.ds(i * L, L)] = fill
    lane = lax.iota(jnp.int32, L)
    last = jnp.full((L,), PAD + L - 1, jnp.int32)

    def body(g, bases):
        tev = te_v[pl.ds(g * L, L)]
        m0 = tev == wid * 2
        m1 = tev == wid * 2 + 1

        def compact(bases):
            b0, b1 = bases
            tok = g * L + lane
            acc = (jnp.where(m0, ones, zeros)
                   + jnp.where(m1, jnp.full((L,), 256, jnp.int32), zeros))
            for k in (1, 2, 4, 8):
                pad_v[pl.ds(PAD, L)] = acc
                acc = acc + plsc.load_gather(pad_v, [lane + (PAD - k)])
            pos0 = (acc & 255) - 1 + b0
            pos1 = (acc >> 8) - 1 + b1
            ok0 = m0 & (pos0 < CAP)
            ok1 = m1 & (pos1 < CAP)
            idx = jnp.where(ok0, pos0, jnp.where(ok1, CAP + pos1, zeros))
            plsc.store_scatter(stt_v, [idx], tok, mask=ok0 | ok1)
            pad_v[pl.ds(PAD, L)] = acc
            tot = plsc.load_gather(pad_v, [last])
            return b0 + (tot & 255), b1 + (tot >> 8)

        return lax.cond(jnp.any(m0 | m1), compact, lambda b: b, bases)

    lax.fori_loop(0, NUM_TOKENS // L, body, (zeros, zeros))
    pltpu.sync_copy(stt_v, stt_hbm.at[pl.ds(wid * 2 * CAP, 2 * CAP)])


def _sc_compact(top_e):
    mesh = plsc.VectorSubcoreMesh(core_axis_name="c", subcore_axis_name="s")
    f = pl.kernel(
        _compact_body,
        mesh=mesh,
        compiler_params=pltpu.CompilerParams(needs_layout_passes=False),
        out_type=jax.ShapeDtypeStruct((NUM_EXPERTS * CAP,), jnp.int32),
        scratch_types=[
            pltpu.VMEM((NUM_TOKENS,), jnp.int32),
            pltpu.VMEM((2 * CAP,), jnp.int32),
            pltpu.VMEM((PAD_LEN,), jnp.int32),
        ],
    )
    return f(top_e)


def _moe_body(idx_ref, wts_ref, x_ref, w1_ref, b1_ref, w2_ref, b2_ref,
              out_ref, xe_ref):
    e = pl.program_id(0)

    @pl.when(e == 0)
    def _init():
        out_ref[...] = jnp.zeros_like(out_ref)

    for c in range(CAP):
        t = idx_ref[e, c]
        ts = jnp.where(t >= NUM_TOKENS, 0, t)
        xe_ref[pl.ds(c, 1), :] = x_ref[pl.ds(ts, 1), :]

    h = jnp.maximum(
        jnp.dot(xe_ref[...], w1_ref[0], preferred_element_type=jnp.float32)
        + b1_ref[0], 0.0)
    y = (jnp.dot(h, w2_ref[0], preferred_element_type=jnp.float32)
         + b2_ref[0])

    for c in range(CAP):
        t = idx_ref[e, c]

        @pl.when(t < NUM_TOKENS)
        def _store():
            out_ref[pl.ds(t, 1), :] = y[c:c + 1, :] * wts_ref[t]


@jax.jit
def kernel(x, w_router, w1, b1, w2, b2, rng):
    T, D = x.shape
    E = NUM_EXPERTS

    logits = x @ w_router
    top_e = jnp.argmax(logits, axis=-1).astype(jnp.int32)
    mx = jnp.max(logits, axis=-1)
    wt = 1.0 / jnp.sum(jnp.exp(logits - mx[:, None]), axis=-1)

    stt = _sc_compact(top_e)
    slot_to_token = stt.reshape(E, CAP)

    grid_spec = pltpu.PrefetchScalarGridSpec(
        num_scalar_prefetch=2,
        grid=(E,),
        in_specs=[
            pl.BlockSpec((T, D), lambda e, *_: (0, 0)),
            pl.BlockSpec((1, D, D_FF), lambda e, *_: (e, 0, 0)),
            pl.BlockSpec((1, 1, D_FF), lambda e, *_: (e, 0, 0)),
            pl.BlockSpec((1, D_FF, D), lambda e, *_: (e, 0, 0)),
            pl.BlockSpec((1, 1, D), lambda e, *_: (e, 0, 0)),
        ],
        out_specs=pl.BlockSpec((T, D), lambda e, *_: (0, 0)),
        scratch_shapes=[pltpu.VMEM((CAP, D), jnp.float32)],
    )
    out = pl.pallas_call(
        _moe_body,
        grid_spec=grid_spec,
        out_shape=jax.ShapeDtypeStruct((T, D), x.dtype),
        compiler_params=pltpu.CompilerParams(
            dimension_semantics=("arbitrary",),
        ),
    )(slot_to_token, wt, x, w1, b1.reshape(E, 1, D_FF), w2,
      b2.reshape(E, 1, D))
    return out
